# baseline (speedup 1.0000x reference)
import jax
import jax.numpy as jnp
from jax import lax
from jax.experimental import pallas as pl
from jax.experimental.pallas import tpu as pltpu
from jax.experimental.pallas import tpu_sc as plsc

NUM_FEATURES = 32
BATCH = 16384

NC = 2
NS = 16
NW = NC * NS
LANES = 16

B_PER_W = BATCH // NW
GROUP = 128
N_GROUPS = B_PER_W // GROUP


def _mf_body(u_hbm, v_hbm, ue_t, ie_t, out_hbm,
             uidx, vidx, ubuf0, ubuf1, vbuf0, vbuf1, out_v, sem0, sem1):
    wid = lax.axis_index("s") * NC + lax.axis_index("c")
    base = wid * B_PER_W

    ubufs = (ubuf0, ubuf1)
    vbufs = (vbuf0, vbuf1)
    sems = (sem0, sem1)

    for j in range(N_GROUPS):
        pltpu.sync_copy(u_hbm.at[pl.ds(base + j * GROUP, GROUP)], uidx.at[j])
        pltpu.sync_copy(v_hbm.at[pl.ds(base + j * GROUP, GROUP)], vidx.at[j])

    def fire(j):
        p = j % 2
        copies = []
        for f in range(NUM_FEATURES):
            copies.append(pltpu.async_copy(
                ue_t.at[f].at[uidx.at[j]], ubufs[p].at[f], sems[p]))
            copies.append(pltpu.async_copy(
                ie_t.at[f].at[vidx.at[j]], vbufs[p].at[f], sems[p]))
        return copies

    def compute(j):
        p = j % 2
        ub, vb = ubufs[p], vbufs[p]

        def cchunk(c, carry):
            acc = jnp.zeros((LANES,), jnp.float32)
            for f in range(NUM_FEATURES):
                a = ub[f, pl.ds(c * LANES, LANES)]
                b = vb[f, pl.ds(c * LANES, LANES)]
                acc = acc + a * b
            out_v[pl.ds(j * GROUP + c * LANES, LANES)] = acc
            return carry

        lax.fori_loop(0, GROUP // LANES, cchunk, 0)

    inflight = {0: fire(0)}
    for j in range(N_GROUPS):
        if j + 1 < N_GROUPS:
            inflight[j + 1] = fire(j + 1)
        for c in inflight.pop(j):
            c.wait()
        compute(j)

    pltpu.sync_copy(out_v, out_hbm.at[pl.ds(base, B_PER_W)])


@jax.jit
def kernel(u, v, user_embed, item_embed):
    mesh = plsc.VectorSubcoreMesh(core_axis_name="c", subcore_axis_name="s")
    f = pl.kernel(
        _mf_body,
        out_type=jax.ShapeDtypeStruct((BATCH,), jnp.float32),
        mesh=mesh,
        scratch_types=[
            pltpu.VMEM((N_GROUPS, GROUP), jnp.int32),
            pltpu.VMEM((N_GROUPS, GROUP), jnp.int32),
            pltpu.VMEM((NUM_FEATURES, GROUP), jnp.float32),
            pltpu.VMEM((NUM_FEATURES, GROUP), jnp.float32),
            pltpu.VMEM((NUM_FEATURES, GROUP), jnp.float32),
            pltpu.VMEM((NUM_FEATURES, GROUP), jnp.float32),
            pltpu.VMEM((B_PER_W,), jnp.float32),
            pltpu.SemaphoreType.DMA,
            pltpu.SemaphoreType.DMA,
        ],
        compiler_params=pltpu.CompilerParams(use_tc_tiling_on_sc=False),
    )
    return f(u, v, user_embed.T, item_embed.T)


# speedup vs baseline: 19.1564x; 19.1564x over previous
import jax
import jax.numpy as jnp
from jax import lax
from jax.experimental import pallas as pl
from jax.experimental.pallas import tpu as pltpu
from jax.experimental.pallas import tpu_sc as plsc

NUM_FEATURES = 32
BATCH = 16384

NC = 2
NS = 16
NW = NC * NS
LANES = 16

B_PER_W = BATCH // NW
N_GROUPS = B_PER_W // LANES
RING = 8


def _mf_body(u_hbm, v_hbm, ue_t, ie_t, out_hbm,
             uidx, vidx, ublk, vblk, out_v, semu, semv):
    wid = lax.axis_index("s") * NC + lax.axis_index("c")
    base = wid * B_PER_W

    for j in range(B_PER_W // 128):
        pltpu.sync_copy(u_hbm.at[pl.ds(base + j * 128, 128)], uidx.at[j])
        pltpu.sync_copy(v_hbm.at[pl.ds(base + j * 128, 128)], vidx.at[j])

    f_lo = lax.iota(jnp.int32, LANES)
    f_hi = f_lo + LANES

    def dot_one(lane_u, lane_v, slot):
        su = jnp.full((LANES,), slot, jnp.int32)
        lu = jnp.full((LANES,), lane_u, jnp.int32)
        lv = jnp.full((LANES,), lane_v, jnp.int32)
        gu_lo = plsc.load_gather(ublk, [su, f_lo, lu])
        gu_hi = plsc.load_gather(ublk, [su, f_hi, lu])
        gv_lo = plsc.load_gather(vblk, [su, f_lo, lv])
        gv_hi = plsc.load_gather(vblk, [su, f_hi, lv])
        return jnp.sum(gu_lo * gv_lo + gu_hi * gv_hi)

    def group(g, carry):
        ru = uidx[g // 8, pl.ds((g % 8) * LANES, LANES)]
        rv = vidx[g // 8, pl.ds((g % 8) * LANES, LANES)]
        acc = jnp.zeros((LANES,), jnp.float32)
        for h in range(2):
            lanes_uv = []
            copies = []
            for k in range(RING):
                l = h * RING + k
                r_u = jax.lax.index_in_dim(ru, l, axis=0, keepdims=False)
                r_v = jax.lax.index_in_dim(rv, l, axis=0, keepdims=False)
                start_u = pl.multiple_of((r_u >> 7) << 7, 128)
                start_v = pl.multiple_of((r_v >> 7) << 7, 128)
                lanes_uv.append((r_u & 127, r_v & 127))
                copies.append(pltpu.async_copy(
                    ue_t.at[:, pl.ds(start_u, 128)], ublk.at[k], semu))
                copies.append(pltpu.async_copy(
                    ie_t.at[:, pl.ds(start_v, 128)], vblk.at[k], semv))
            for c in copies:
                c.wait()
            for k, (lane_u, lane_v) in enumerate(lanes_uv):
                tot = dot_one(lane_u, lane_v, k)
                sel = f_lo == (h * RING + k)
                acc = jnp.where(sel, jnp.full((LANES,), tot), acc)
        out_v[pl.ds(g * LANES, LANES)] = acc
        return carry

    lax.fori_loop(0, N_GROUPS, group, 0)

    pltpu.sync_copy(out_v, out_hbm.at[pl.ds(base, B_PER_W)])


@jax.jit
def kernel(u, v, user_embed, item_embed):
    mesh = plsc.VectorSubcoreMesh(core_axis_name="c", subcore_axis_name="s")
    f = pl.kernel(
        _mf_body,
        out_type=jax.ShapeDtypeStruct((BATCH,), jnp.float32),
        mesh=mesh,
        scratch_types=[
            pltpu.VMEM((B_PER_W // 128, 128), jnp.int32),
            pltpu.VMEM((B_PER_W // 128, 128), jnp.int32),
            pltpu.VMEM((RING, NUM_FEATURES, 128), jnp.float32),
            pltpu.VMEM((RING, NUM_FEATURES, 128), jnp.float32),
            pltpu.VMEM((B_PER_W,), jnp.float32),
            pltpu.SemaphoreType.DMA,
            pltpu.SemaphoreType.DMA,
        ],
        compiler_params=pltpu.CompilerParams(needs_layout_passes=False),
    )
    return f(u, v, user_embed.T, item_embed.T)
